# baseline (speedup 1.0000x reference)
import jax
import jax.numpy as jnp
from jax import lax
from jax.experimental import pallas as pl
from jax.experimental.pallas import tpu as pltpu
from jax.experimental.pallas import tpu_sc as plsc

H, W = 180, 240
K = 10
R = 3
S = 2 * R + 1
SS = S * S
TAU = 1000000.0
DELTA_T = 100000.0
GH, GW = H // K, W // K
NCELLS = GH * GW
B, T = 8, 4096
C = 16
HP, WP = H + 2 * R, W + 2 * R
MEMN = ((HP * WP + C - 1) // C) * C
CNTN = ((NCELLS + C - 1) // C) * C
HN = NCELLS * SS
NEG = -1.0e18


def _hats_body(ev_hbm, len_hbm, out_hbm,
               ebuf, xv, yv, tv, lenv, mem, hist, cnt, neigh, loserb, sib,
               shared):
    pol = lax.axis_index("c")
    sid = lax.axis_index("s")
    b = sid & 7
    half = sid >> 3

    pltpu.sync_copy(ev_hbm.at[b], ebuf)
    pltpu.sync_copy(len_hbm, lenv)

    iotai = lax.iota(jnp.int32, C)
    negv = jnp.full((C,), NEG, jnp.float32)
    zerov = jnp.zeros((C,), jnp.float32)
    onesv = jnp.ones((C,), jnp.float32)
    polf = pol.astype(jnp.float32)

    def initmem(i, carry):
        base = i * 4 * C
        for u in range(4):
            mem[pl.ds(base + u * C, C)] = negv
        return carry
    lax.fori_loop(0, MEMN // (4 * C), initmem, 0)

    def inithist(i, carry):
        base = i * 4 * C
        for u in range(4):
            hist[pl.ds(base + u * C, C)] = zerov
        return carry
    lax.fori_loop(0, HN // (4 * C), inithist, 0)

    def inithist_tail(i, carry):
        hist[pl.ds((HN // (4 * C)) * 4 * C + i * C, C)] = zerov
        return carry
    lax.fori_loop(0, (HN % (4 * C)) // C, inithist_tail, 0)

    def initcnt(i, carry):
        cnt[pl.ds(i * C, C)] = zerov
        return carry
    lax.fori_loop(0, CNTN // C, initcnt, 0)

    length = lenv[pl.ds(b, C)][0]
    nch = (length + (C - 1)) >> 4

    def compact_body(ci, pos):
        base = ci * C
        iv = base + iotai
        i4 = iv * 4
        xf = plsc.load_gather(ebuf, [i4])
        yf = plsc.load_gather(ebuf, [i4 + 1])
        tf = plsc.load_gather(ebuf, [i4 + 2])
        pf = plsc.load_gather(ebuf, [i4 + 3])
        mk = (iv < length) & (pf == polf)
        plsc.store_compressed(xv.at[pl.ds(pos, C)], xf, mask=mk)
        plsc.store_compressed(yv.at[pl.ds(pos, C)], yf, mask=mk)
        plsc.store_compressed(tv.at[pl.ds(pos, C)], tf, mask=mk)
        return pos + plsc.all_reduce_population_count(mk)[0]
    npol = lax.fori_loop(0, nch, compact_body, 0)
    nch2 = (npol + (C - 1)) >> 4

    imid = npol >> 1

    tmid = tv[pl.ds(imid, C)][0]
    tlimit = tmid - DELTA_T

    def bs_body(i, lohi):
        lo, hi = lohi
        mid = (lo + hi) >> 1
        v = tv[pl.ds(mid, C)][0]
        pred = v >= tlimit
        return (jnp.where(pred, lo, mid + 1), jnp.where(pred, mid, hi))
    j0, _ = lax.fori_loop(0, 12, bs_body, (jnp.int32(0), imid))
    j0a = j0 & ~(C - 1)
    nrep = jnp.where(half == 1, (imid - j0a + (C - 1)) >> 4, 0)

    def replay_body(ci, carry):
        base = j0a + ci * C
        iv = base + iotai
        xf = xv[pl.ds(base, C)]
        yf = yv[pl.ds(base, C)]
        tf = tv[pl.ds(base, C)]
        mrep = iv < imid
        loserb[pl.ds(0, C)] = zerov
        for s in range(C - 1, 0, -1):
            jidx = jnp.maximum(iv - s, 0)
            xj = plsc.load_gather(xv, [jidx])
            yj = plsc.load_gather(yv, [jidx])
            exact = ((iotai >= s) & (xj == xf) & (yj == yf) & mrep)
            laddr = jnp.maximum(iotai - s, 0)
            plsc.store_scatter(loserb, [laddr], onesv, mask=exact)
        lz = loserb[pl.ds(0, C)]
        winners = mrep & (lz == 0.0)
        xi = xf.astype(jnp.int32)
        yi = yf.astype(jnp.int32)
        pidx = (yi + R) * WP + (xi + R)
        pidx = jnp.clip(pidx, R * WP + R, (H + R - 1) * WP + W + R - 1)
        plsc.store_scatter(mem, [pidx], tf, mask=winners)
        return carry
    lax.fori_loop(0, nrep, replay_body, 0)

    lo_b = jnp.where(half == 1, imid, 0)
    hi_b = jnp.where(half == 1, npol, imid)
    first = jnp.where(half == 1, imid >> 4, 0)
    count = jnp.where(half == 1, nch2 - (imid >> 4), (imid + (C - 1)) >> 4)

    def chunk_body(ci, carry):
        base = (first + ci) * C
        iv = base + iotai
        xf = xv[pl.ds(base, C)]
        yf = yv[pl.ds(base, C)]
        tf = tv[pl.ds(base, C)]
        xi = xf.astype(jnp.int32)
        yi = yf.astype(jnp.int32)
        m = (iv >= lo_b) & (iv < hi_b)
        mf = jnp.where(m, 1.0, 0.0)
        pidx = (yi + R) * WP + (xi + R)
        pidx = jnp.clip(pidx, R * WP + R, (H + R - 1) * WP + W + R - 1)

        for o in range(SS):
            if o == SS // 2:
                continue
            doff = (o // S - R) * WP + (o % S - R)
            g = plsc.load_gather(mem, [pidx + doff])
            neigh[pl.ds(o * C, C)] = g

        loserb[pl.ds(0, C)] = zerov
        for s in range(C - 1, 0, -1):
            jidx = jnp.maximum(iv - s, 0)
            xj = plsc.load_gather(xv, [jidx])
            yj = plsc.load_gather(yv, [jidx])
            tj = plsc.load_gather(tv, [jidx])
            mj = iotai >= s
            dxf = xj - xf
            dyf = yj - yf
            match = mj & (jnp.abs(dxf) <= 3.0) & (jnp.abs(dyf) <= 3.0)
            of = jnp.clip((dyf + 3.0) * 7.0 + (dxf + 3.0), 0.0, 48.0)
            addr = of.astype(jnp.int32) * C + iotai
            plsc.store_scatter(neigh, [addr], tj, mask=match)
            exact = match & (dxf == 0.0) & (dyf == 0.0) & m
            laddr = jnp.maximum(iotai - s, 0)
            plsc.store_scatter(loserb, [laddr], onesv, mask=exact)

        chv = ((yf + 0.5) * (1.0 / K)).astype(jnp.int32)
        cwv = ((xf + 0.5) * (1.0 / K)).astype(jnp.int32)
        cidv = jnp.clip(chv * GW + cwv, 0, NCELLS - 1)
        hbase = cidv * SS
        for o in range(SS):
            if o == SS // 2:
                sv = mf
            else:
                g = neigh[pl.ds(o * C, C)]
                dt = tf - g
                win = dt <= DELTA_T
                e = jnp.exp(dt * (-1.0 / TAU))
                sv = jnp.where(win & m, e, 0.0)
            plsc.addupdate_scatter(hist, [hbase + o], sv)
        plsc.addupdate_scatter(cnt, [cidv], mf)

        lz = loserb[pl.ds(0, C)]
        winners = m & (lz == 0.0)
        plsc.store_scatter(mem, [pidx], tf, mask=winners)
        return carry
    lax.fori_loop(0, count, chunk_body, 0)

    shbase = pl.multiple_of(b * (HN + CNTN), 8)

    @pl.when(half == 1)
    def _():
        pltpu.sync_copy(hist, shared.at[pl.ds(shbase, HN)])
        pltpu.sync_copy(cnt, shared.at[pl.ds(shbase + HN, CNTN)])
    plsc.subcore_barrier()

    @pl.when(half == 0)
    def _():
        pltpu.sync_copy(shared.at[pl.ds(shbase, HN + CNTN)], sib)

        def merge_cnt(i, carry):
            slc = pl.ds(i * C, C)
            cnt[slc] = cnt[slc] + sib[pl.ds(HN + i * C, C)]
            return carry
        lax.fori_loop(0, CNTN // C, merge_cnt, 0)

        def norm_body(i, carry):
            for u in range(3):
                fv = (i * 3 + u) * C + iotai
                cf = ((fv.astype(jnp.float32) + 0.5)
                      * (1.0 / SS)).astype(jnp.int32)
                dv = plsc.load_gather(cnt, [cf])
                hslc = pl.ds((i * 3 + u) * C, C)
                hist[hslc] = ((hist[hslc] + sib[hslc])
                              / jnp.maximum(dv, 1.0))
            return carry
        lax.fori_loop(0, HN // (3 * C), norm_body, 0)
        pltpu.sync_copy(hist, out_hbm.at[b, pol])


_hats_call = pl.kernel(
    _hats_body,
    out_type=jax.ShapeDtypeStruct((B, 2, HN), jnp.float32),
    mesh=plsc.VectorSubcoreMesh(core_axis_name="c", subcore_axis_name="s"),
    compiler_params=pltpu.CompilerParams(needs_layout_passes=False),
    scratch_types=[
        pltpu.VMEM((T * 4,), jnp.float32),
        pltpu.VMEM((T + C,), jnp.float32),
        pltpu.VMEM((T + C,), jnp.float32),
        pltpu.VMEM((T + C,), jnp.float32),
        pltpu.VMEM((32,), jnp.int32),
        pltpu.VMEM((MEMN,), jnp.float32),
        pltpu.VMEM((HN,), jnp.float32),
        pltpu.VMEM((CNTN,), jnp.float32),
        pltpu.VMEM((SS * C,), jnp.float32),
        pltpu.VMEM((C,), jnp.float32),
        pltpu.VMEM((HN + CNTN,), jnp.float32),
        pltpu.VMEM_SHARED((B * (HN + CNTN),), jnp.float32),
    ],
)


def kernel(events, lengths):
    len_pad = jnp.concatenate(
        [lengths.astype(jnp.int32), jnp.zeros((32 - B,), jnp.int32)])
    out = _hats_call(events.reshape(B, T * 4), len_pad)
    return out.reshape(B, 2, NCELLS, S, S).transpose(0, 2, 1, 3, 4)


# speedup vs baseline: 1.0461x; 1.0461x over previous
import jax
import jax.numpy as jnp
from jax import lax
from jax.experimental import pallas as pl
from jax.experimental.pallas import tpu as pltpu
from jax.experimental.pallas import tpu_sc as plsc

H, W = 180, 240
K = 10
R = 3
S = 2 * R + 1
SS = S * S
TAU = 1000000.0
DELTA_T = 100000.0
GH, GW = H // K, W // K
NCELLS = GH * GW
B, T = 8, 4096
C = 16
HP, WP = H + 2 * R, W + 2 * R
MEMN = ((HP * WP + C - 1) // C) * C
CNTN = ((NCELLS + C - 1) // C) * C
HN = NCELLS * SS
NEG = -1.0e18


def _hats_body(x_hbm, y_hbm, t_hbm, p_hbm, len_hbm, out_hbm,
               xv, yv, tv, pv, lenv, mem, hist, cnt, neigh, loserb, sib,
               shared):
    pol = lax.axis_index("c")
    sid = lax.axis_index("s")
    b = sid & 7
    half = sid >> 3

    pltpu.sync_copy(x_hbm.at[b], xv.at[pl.ds(0, T)])
    pltpu.sync_copy(y_hbm.at[b], yv.at[pl.ds(0, T)])
    pltpu.sync_copy(t_hbm.at[b], tv.at[pl.ds(0, T)])
    pltpu.sync_copy(p_hbm.at[b], pv)
    pltpu.sync_copy(len_hbm, lenv)

    iotai = lax.iota(jnp.int32, C)
    negv = jnp.full((C,), NEG, jnp.float32)
    zerov = jnp.zeros((C,), jnp.float32)
    onesv = jnp.ones((C,), jnp.float32)
    polf = pol.astype(jnp.float32)

    def initmem(i, carry):
        base = i * 4 * C
        for u in range(4):
            mem[pl.ds(base + u * C, C)] = negv
        return carry
    lax.fori_loop(0, MEMN // (4 * C), initmem, 0)

    def inithist(i, carry):
        base = i * 4 * C
        for u in range(4):
            hist[pl.ds(base + u * C, C)] = zerov
        return carry
    lax.fori_loop(0, HN // (4 * C), inithist, 0)

    def inithist_tail(i, carry):
        hist[pl.ds((HN // (4 * C)) * 4 * C + i * C, C)] = zerov
        return carry
    lax.fori_loop(0, (HN % (4 * C)) // C, inithist_tail, 0)

    def initcnt(i, carry):
        cnt[pl.ds(i * C, C)] = zerov
        return carry
    lax.fori_loop(0, CNTN // C, initcnt, 0)

    length = lenv[pl.ds(b, C)][0]
    nch = (length + (C - 1)) >> 4

    def compact_body(ci, pos):
        base = ci * C
        iv = base + iotai
        xf = xv[pl.ds(base, C)]
        yf = yv[pl.ds(base, C)]
        tf = tv[pl.ds(base, C)]
        pf = pv[pl.ds(base, C)]
        mk = (iv < length) & (pf == polf)
        plsc.store_compressed(xv.at[pl.ds(pos, C)], xf, mask=mk)
        plsc.store_compressed(yv.at[pl.ds(pos, C)], yf, mask=mk)
        plsc.store_compressed(tv.at[pl.ds(pos, C)], tf, mask=mk)
        return pos + plsc.all_reduce_population_count(mk)[0]
    npol = lax.fori_loop(0, nch, compact_body, 0)
    nch2 = (npol + (C - 1)) >> 4

    imid = npol >> 1

    tmid = tv[pl.ds(imid, C)][0]
    tlimit = tmid - DELTA_T

    def bs_body(i, lohi):
        lo, hi = lohi
        mid = (lo + hi) >> 1
        v = tv[pl.ds(mid, C)][0]
        pred = v >= tlimit
        return (jnp.where(pred, lo, mid + 1), jnp.where(pred, mid, hi))
    j0, _ = lax.fori_loop(0, 12, bs_body, (jnp.int32(0), imid))
    j0a = j0 & ~(C - 1)
    nrep = jnp.where(half == 1, (imid - j0a + (C - 1)) >> 4, 0)

    def replay_body(ci, carry):
        base = j0a + ci * C
        iv = base + iotai
        xf = xv[pl.ds(base, C)]
        yf = yv[pl.ds(base, C)]
        tf = tv[pl.ds(base, C)]
        mrep = iv < imid
        loserb[pl.ds(0, C)] = zerov
        for s in range(C - 1, 0, -1):
            jidx = jnp.maximum(iv - s, 0)
            xj = plsc.load_gather(xv, [jidx])
            yj = plsc.load_gather(yv, [jidx])
            exact = ((iotai >= s) & (xj == xf) & (yj == yf) & mrep)
            laddr = jnp.maximum(iotai - s, 0)
            plsc.store_scatter(loserb, [laddr], onesv, mask=exact)
        lz = loserb[pl.ds(0, C)]
        winners = mrep & (lz == 0.0)
        xi = xf.astype(jnp.int32)
        yi = yf.astype(jnp.int32)
        pidx = (yi + R) * WP + (xi + R)
        plsc.store_scatter(mem, [pidx], tf, mask=winners)
        return carry
    lax.fori_loop(0, nrep, replay_body, 0)

    lo_b = jnp.where(half == 1, imid, 0)
    hi_b = jnp.where(half == 1, npol, imid)
    first = jnp.where(half == 1, imid >> 4, 0)
    count = jnp.where(half == 1, nch2 - (imid >> 4), (imid + (C - 1)) >> 4)

    def chunk_body(ci, carry):
        base = (first + ci) * C
        iv = base + iotai
        xf = xv[pl.ds(base, C)]
        yf = yv[pl.ds(base, C)]
        tf = tv[pl.ds(base, C)]
        xi = xf.astype(jnp.int32)
        yi = yf.astype(jnp.int32)
        m = (iv >= lo_b) & (iv < hi_b)
        mf = jnp.where(m, 1.0, 0.0)
        pidx = (yi + R) * WP + (xi + R)

        for o in range(SS):
            if o == SS // 2:
                continue
            doff = (o // S - R) * WP + (o % S - R)
            g = plsc.load_gather(mem, [pidx + doff])
            neigh[pl.ds(o * C, C)] = g

        loserb[pl.ds(0, C)] = zerov
        for s in range(C - 1, 0, -1):
            jidx = jnp.maximum(iv - s, 0)
            xj = plsc.load_gather(xv, [jidx])
            yj = plsc.load_gather(yv, [jidx])
            tj = plsc.load_gather(tv, [jidx])
            mj = iotai >= s
            dxf = xj - xf
            dyf = yj - yf
            match = mj & (jnp.abs(dxf) <= 3.0) & (jnp.abs(dyf) <= 3.0)
            of = jnp.clip((dyf + 3.0) * 7.0 + (dxf + 3.0), 0.0, 48.0)
            addr = of.astype(jnp.int32) * C + iotai
            plsc.store_scatter(neigh, [addr], tj, mask=match)
            exact = match & (dxf == 0.0) & (dyf == 0.0) & m
            laddr = jnp.maximum(iotai - s, 0)
            plsc.store_scatter(loserb, [laddr], onesv, mask=exact)

        chv = ((yf + 0.5) * (1.0 / K)).astype(jnp.int32)
        cwv = ((xf + 0.5) * (1.0 / K)).astype(jnp.int32)
        cidv = jnp.clip(chv * GW + cwv, 0, NCELLS - 1)
        hbase = cidv * SS
        for o in range(SS):
            if o == SS // 2:
                sv = mf
            else:
                g = neigh[pl.ds(o * C, C)]
                dt = tf - g
                win = dt <= DELTA_T
                e = jnp.exp(dt * (-1.0 / TAU))
                sv = jnp.where(win & m, e, 0.0)
            plsc.addupdate_scatter(hist, [hbase + o], sv)
        plsc.addupdate_scatter(cnt, [cidv], mf)

        lz = loserb[pl.ds(0, C)]
        winners = m & (lz == 0.0)
        plsc.store_scatter(mem, [pidx], tf, mask=winners)
        return carry
    lax.fori_loop(0, count, chunk_body, 0)

    shbase = pl.multiple_of(b * (HN + CNTN), 8)

    @pl.when(half == 1)
    def _():
        pltpu.sync_copy(hist, shared.at[pl.ds(shbase, HN)])
        pltpu.sync_copy(cnt, shared.at[pl.ds(shbase + HN, CNTN)])
    plsc.subcore_barrier()

    @pl.when(half == 0)
    def _():
        pltpu.sync_copy(shared.at[pl.ds(shbase, HN + CNTN)], sib)

        def merge_cnt(i, carry):
            slc = pl.ds(i * C, C)
            cnt[slc] = cnt[slc] + sib[pl.ds(HN + i * C, C)]
            return carry
        lax.fori_loop(0, CNTN // C, merge_cnt, 0)

        def norm_body(i, carry):
            for u in range(3):
                fv = (i * 3 + u) * C + iotai
                cf = ((fv.astype(jnp.float32) + 0.5)
                      * (1.0 / SS)).astype(jnp.int32)
                dv = plsc.load_gather(cnt, [cf])
                hslc = pl.ds((i * 3 + u) * C, C)
                hist[hslc] = ((hist[hslc] + sib[hslc])
                              / jnp.maximum(dv, 1.0))
            return carry
        lax.fori_loop(0, HN // (3 * C), norm_body, 0)
        pltpu.sync_copy(hist, out_hbm.at[b, pol])


_hats_call = pl.kernel(
    _hats_body,
    out_type=jax.ShapeDtypeStruct((B, 2, HN), jnp.float32),
    mesh=plsc.VectorSubcoreMesh(core_axis_name="c", subcore_axis_name="s"),
    compiler_params=pltpu.CompilerParams(needs_layout_passes=False),
    scratch_types=[
        pltpu.VMEM((T + C,), jnp.float32),
        pltpu.VMEM((T + C,), jnp.float32),
        pltpu.VMEM((T + C,), jnp.float32),
        pltpu.VMEM((T,), jnp.float32),
        pltpu.VMEM((32,), jnp.int32),
        pltpu.VMEM((MEMN,), jnp.float32),
        pltpu.VMEM((HN,), jnp.float32),
        pltpu.VMEM((CNTN,), jnp.float32),
        pltpu.VMEM((SS * C,), jnp.float32),
        pltpu.VMEM((C,), jnp.float32),
        pltpu.VMEM((HN + CNTN,), jnp.float32),
        pltpu.VMEM_SHARED((B * (HN + CNTN),), jnp.float32),
    ],
)


def kernel(events, lengths):
    x = events[..., 0]
    y = events[..., 1]
    t = events[..., 2]
    p = events[..., 3]
    len_pad = jnp.concatenate(
        [lengths.astype(jnp.int32), jnp.zeros((32 - B,), jnp.int32)])
    out = _hats_call(x, y, t, p, len_pad)
    return out.reshape(B, 2, NCELLS, S, S).transpose(0, 2, 1, 3, 4)


# speedup vs baseline: 1.0470x; 1.0009x over previous
import jax
import jax.numpy as jnp
from jax import lax
from jax.experimental import pallas as pl
from jax.experimental.pallas import tpu as pltpu
from jax.experimental.pallas import tpu_sc as plsc

H, W = 180, 240
K = 10
R = 3
S = 2 * R + 1
SS = S * S
TAU = 1000000.0
DELTA_T = 100000.0
GH, GW = H // K, W // K
NCELLS = GH * GW
B, T = 8, 4096
C = 16
HP, WP = H + 2 * R, W + 2 * R
MEMN = ((HP * WP + C - 1) // C) * C
CNTN = ((NCELLS + C - 1) // C) * C
HN = NCELLS * SS
NEG = -1.0e18


def _hats_body(x_hbm, y_hbm, t_hbm, p_hbm, len_hbm, out_hbm,
               xv, yv, tv, pv, lenv, mem, hist, cnt, neigh, loserb, sib,
               shared):
    pol = lax.axis_index("c")
    sid = lax.axis_index("s")
    b = sid & 7
    half = sid >> 3

    pltpu.sync_copy(x_hbm.at[b], xv.at[pl.ds(0, T)])
    pltpu.sync_copy(y_hbm.at[b], yv.at[pl.ds(0, T)])
    pltpu.sync_copy(t_hbm.at[b], tv.at[pl.ds(0, T)])
    pltpu.sync_copy(p_hbm.at[b], pv)
    pltpu.sync_copy(len_hbm, lenv)

    iotai = lax.iota(jnp.int32, C)
    negv = jnp.full((C,), NEG, jnp.float32)
    zerov = jnp.zeros((C,), jnp.float32)
    onesv = jnp.ones((C,), jnp.float32)
    polf = pol.astype(jnp.float32)

    def initmem(i, carry):
        base = i * 4 * C
        for u in range(4):
            mem[pl.ds(base + u * C, C)] = negv
        return carry
    lax.fori_loop(0, MEMN // (4 * C), initmem, 0)

    def inithist(i, carry):
        base = i * 4 * C
        for u in range(4):
            hist[pl.ds(base + u * C, C)] = zerov
        return carry
    lax.fori_loop(0, HN // (4 * C), inithist, 0)

    def inithist_tail(i, carry):
        hist[pl.ds((HN // (4 * C)) * 4 * C + i * C, C)] = zerov
        return carry
    lax.fori_loop(0, (HN % (4 * C)) // C, inithist_tail, 0)

    def initcnt(i, carry):
        cnt[pl.ds(i * C, C)] = zerov
        return carry
    lax.fori_loop(0, CNTN // C, initcnt, 0)

    length = lenv[pl.ds(b, C)][0]
    nch = (length + (C - 1)) >> 4

    def compact_body(ci, pos):
        base = ci * C
        iv = base + iotai
        xf = xv[pl.ds(base, C)]
        yf = yv[pl.ds(base, C)]
        tf = tv[pl.ds(base, C)]
        pf = pv[pl.ds(base, C)]
        mk = (iv < length) & (pf == polf)
        plsc.store_compressed(xv.at[pl.ds(pos, C)], xf, mask=mk)
        plsc.store_compressed(yv.at[pl.ds(pos, C)], yf, mask=mk)
        plsc.store_compressed(tv.at[pl.ds(pos, C)], tf, mask=mk)
        return pos + plsc.all_reduce_population_count(mk)[0]
    npol = lax.fori_loop(0, nch, compact_body, 0)
    nch2 = (npol + (C - 1)) >> 4

    imid = npol >> 1

    tmid = tv[pl.ds(imid, C)][0]
    tlimit = tmid - DELTA_T

    def bs_body(i, lohi):
        lo, hi = lohi
        mid = (lo + hi) >> 1
        v = tv[pl.ds(mid, C)][0]
        pred = v >= tlimit
        return (jnp.where(pred, lo, mid + 1), jnp.where(pred, mid, hi))
    j0, _ = lax.fori_loop(0, 12, bs_body, (jnp.int32(0), imid))
    j0a = j0 & ~(C - 1)
    nrep = jnp.where(half == 1, (imid - j0a + (C - 1)) >> 4, 0)

    def replay_body(ci, carry):
        base = j0a + ci * C
        iv = base + iotai
        xf = xv[pl.ds(base, C)]
        yf = yv[pl.ds(base, C)]
        tf = tv[pl.ds(base, C)]
        mrep = iv < imid
        loserb[pl.ds(0, C)] = zerov
        for s in range(C - 1, 0, -1):
            jidx = jnp.maximum(iv - s, 0)
            xj = plsc.load_gather(xv, [jidx])
            yj = plsc.load_gather(yv, [jidx])
            exact = ((iotai >= s) & (xj == xf) & (yj == yf) & mrep)
            laddr = jnp.maximum(iotai - s, 0)
            plsc.store_scatter(loserb, [laddr], onesv, mask=exact)
        lz = loserb[pl.ds(0, C)]
        winners = mrep & (lz == 0.0)
        xi = xf.astype(jnp.int32)
        yi = yf.astype(jnp.int32)
        pidx = (yi + R) * WP + (xi + R)
        plsc.store_scatter(mem, [pidx], tf, mask=winners)
        return carry
    lax.fori_loop(0, nrep, replay_body, 0)

    lo_b = jnp.where(half == 1, imid, 0)
    hi_b = jnp.where(half == 1, npol, imid)
    first = jnp.where(half == 1, imid >> 4, 0)
    count = jnp.where(half == 1, nch2 - (imid >> 4), (imid + (C - 1)) >> 4)

    def chunk_body(ci, carry):
        base = (first + ci) * C
        iv = base + iotai
        xf = xv[pl.ds(base, C)]
        yf = yv[pl.ds(base, C)]
        tf = tv[pl.ds(base, C)]
        xi = xf.astype(jnp.int32)
        yi = yf.astype(jnp.int32)
        m = (iv >= lo_b) & (iv < hi_b)
        mf = jnp.where(m, 1.0, 0.0)
        pidx = (yi + R) * WP + (xi + R)

        for o in range(SS):
            if o == SS // 2:
                continue
            doff = (o // S - R) * WP + (o % S - R)
            g = plsc.load_gather(mem, [pidx + doff])
            neigh[pl.ds(o * C, C)] = g

        loserb[pl.ds(0, C)] = zerov
        for s in range(C - 1, 0, -1):
            jidx = jnp.maximum(iv - s, 0)
            xj = plsc.load_gather(xv, [jidx])
            yj = plsc.load_gather(yv, [jidx])
            tj = plsc.load_gather(tv, [jidx])
            mj = iotai >= s
            dxf = xj - xf
            dyf = yj - yf
            match = mj & (jnp.abs(dxf) <= 3.0) & (jnp.abs(dyf) <= 3.0)
            of = jnp.clip((dyf + 3.0) * 7.0 + (dxf + 3.0), 0.0, 48.0)
            addr = of.astype(jnp.int32) * C + iotai
            plsc.store_scatter(neigh, [addr], tj, mask=match)
            exact = match & (dxf == 0.0) & (dyf == 0.0) & m
            laddr = jnp.maximum(iotai - s, 0)
            plsc.store_scatter(loserb, [laddr], onesv, mask=exact)

        chv = ((yf + 0.5) * (1.0 / K)).astype(jnp.int32)
        cwv = ((xf + 0.5) * (1.0 / K)).astype(jnp.int32)
        cidv = jnp.clip(chv * GW + cwv, 0, NCELLS - 1)
        hbase = cidv * SS
        for o in range(SS):
            if o == SS // 2:
                sv = mf
            else:
                g = neigh[pl.ds(o * C, C)]
                dt = tf - g
                win = dt <= DELTA_T
                e = jnp.exp(dt * (-1.0 / TAU))
                sv = jnp.where(win & m, e, 0.0)
            plsc.addupdate_scatter(hist, [hbase + o], sv)
        plsc.addupdate_scatter(cnt, [cidv], mf)

        lz = loserb[pl.ds(0, C)]
        winners = m & (lz == 0.0)
        plsc.store_scatter(mem, [pidx], tf, mask=winners)
        return carry
    lax.fori_loop(0, count, chunk_body, 0)

    shbase = pl.multiple_of(b * (HN + CNTN), 8)

    @pl.when(half == 1)
    def _():
        pltpu.sync_copy(hist, shared.at[pl.ds(shbase, HN)])
        pltpu.sync_copy(cnt, shared.at[pl.ds(shbase + HN, CNTN)])
    plsc.subcore_barrier()

    @pl.when(half == 0)
    def _():
        pltpu.sync_copy(shared.at[pl.ds(shbase, HN + CNTN)], sib)

        def merge_cnt(i, carry):
            slc = pl.ds(i * C, C)
            cnt[slc] = cnt[slc] + sib[pl.ds(HN + i * C, C)]
            return carry
        lax.fori_loop(0, CNTN // C, merge_cnt, 0)

        def norm_body(i, carry):
            for u in range(3):
                fv = (i * 3 + u) * C + iotai
                cf = ((fv.astype(jnp.float32) + 0.5)
                      * (1.0 / SS)).astype(jnp.int32)
                dv = plsc.load_gather(cnt, [cf])
                hslc = pl.ds((i * 3 + u) * C, C)
                hist[hslc] = ((hist[hslc] + sib[hslc])
                              / jnp.maximum(dv, 1.0))
            return carry
        lax.fori_loop(0, HN // (3 * C), norm_body, 0)
        pltpu.sync_copy(hist, out_hbm.at[b, pol])


_hats_call_cache = []


def _hats_call(*args):
    if not _hats_call_cache:
        _hats_call_cache.append(pl.kernel(
            _hats_body,
            out_type=jax.ShapeDtypeStruct((B, 2, HN), jnp.float32),
            mesh=plsc.VectorSubcoreMesh(
                core_axis_name="c", subcore_axis_name="s"),
            compiler_params=pltpu.CompilerParams(needs_layout_passes=False),
            scratch_types=[
                pltpu.VMEM((T + C,), jnp.float32),
                pltpu.VMEM((T + C,), jnp.float32),
                pltpu.VMEM((T + C,), jnp.float32),
                pltpu.VMEM((T,), jnp.float32),
                pltpu.VMEM((32,), jnp.int32),
                pltpu.VMEM((MEMN,), jnp.float32),
                pltpu.VMEM((HN,), jnp.float32),
                pltpu.VMEM((CNTN,), jnp.float32),
                pltpu.VMEM((SS * C,), jnp.float32),
                pltpu.VMEM((C,), jnp.float32),
                pltpu.VMEM((HN + CNTN,), jnp.float32),
                pltpu.VMEM_SHARED((B * (HN + CNTN),), jnp.float32),
            ],
        ))
    return _hats_call_cache[0](*args)


def kernel(events, lengths):
    x = events[..., 0]
    y = events[..., 1]
    t = events[..., 2]
    p = events[..., 3]
    len_pad = jnp.concatenate(
        [lengths.astype(jnp.int32), jnp.zeros((32 - B,), jnp.int32)])
    out = _hats_call(x, y, t, p, len_pad)
    return out.reshape(B, 2, NCELLS, S, S).transpose(0, 2, 1, 3, 4)
